# baseline (speedup 1.0000x reference)
import jax
import jax.numpy as jnp
from jax import lax
from jax.experimental import pallas as pl
from jax.experimental.pallas import tpu as pltpu
from jax.experimental.pallas import tpu_sc as plsc

VOCAB = 1000000
D = 64
B = 4096
S = 200
SA = 128
SB = S - SA
NC = 2
NS = 16
NW = NC * NS
ROWS_PER_W = B // NW
G = 2
NCHUNK = ROWS_PER_W // G
NBUF = 4


def _body(table_hbm, pos2_hbm, idxa_hbm, idxb_hbm, out_hbm,
          idxa_v, idxb_v, buf0, buf1, buf2, buf3,
          p0, p1, p2, p3, g0, g1, g2, g3, o0, o1, o2, o3):
    bufs = (buf0, buf1, buf2, buf3)
    psems = (p0, p1, p2, p3)
    gsems = (g0, g1, g2, g3)
    osems = (o0, o1, o2, o3)

    cid = lax.axis_index("c")
    sid = lax.axis_index("s")
    wid = sid * NC + cid
    base = wid * ROWS_PER_W

    pltpu.sync_copy(idxa_hbm.at[pl.ds(base, ROWS_PER_W)], idxa_v)
    pltpu.sync_copy(idxb_hbm.at[pl.ds(base, ROWS_PER_W)], idxb_v)

    def prefill(s):
        return pltpu.make_async_copy(pos2_hbm, bufs[s], psems[s])

    def gathers(g, s):
        row = G * g
        cps = []
        for r in range(G):
            cps.append(pltpu.make_async_copy(
                table_hbm.at[idxa_v.at[row + r]],
                bufs[s].at[pl.ds(r * S, SA)], gsems[s]))
            cps.append(pltpu.make_async_copy(
                table_hbm.at[idxb_v.at[row + r]],
                bufs[s].at[pl.ds(r * S + SA, SB)], gsems[s]))
        return cps

    def store(g, s):
        return pltpu.make_async_copy(
            bufs[s], out_hbm.at[pl.ds((base + G * g) * S, G * S)], osems[s])

    prefill(0).start()

    def step(gg, carry):
        for s in range(NBUF):
            g = gg * NBUF + s
            sn = (s + 1) % NBUF
            sp = (s + NBUF - 1) % NBUF

            prefill(s).wait()
            for cp in gathers(g, s):
                cp.start(add=True)

            @pl.when(g >= NBUF - 1)
            def _():
                store(jnp.maximum(g - (NBUF - 1), 0), sn).wait()

            @pl.when(g + 1 <= NCHUNK - 1)
            def _():
                prefill(sn).start()

            @pl.when(g >= 1)
            def _():
                gp = jnp.maximum(g - 1, 0)
                for cp in gathers(gp, sp):
                    cp.wait()
                store(gp, sp).start()
        return carry

    lax.fori_loop(0, NCHUNK // NBUF, step, 0)

    last = NCHUNK - 1
    ls = last % NBUF
    for cp in gathers(last, ls):
        cp.wait()
    store(last, ls).start()
    for g in range(NCHUNK - NBUF + 1, NCHUNK):
        store(g, g % NBUF).wait()


def kernel(x, embedding_weight, positional_weight):
    idxa = x[:, :SA].astype(jnp.int32)
    idxb = x[:, SA:].astype(jnp.int32)
    pos2 = jnp.tile(positional_weight[:S], (G, 1))
    mesh = plsc.VectorSubcoreMesh(core_axis_name="c", subcore_axis_name="s")
    out = pl.kernel(
        _body,
        out_type=jax.ShapeDtypeStruct((B * S, D), jnp.float32),
        mesh=mesh,
        scratch_types=[
            pltpu.VMEM((ROWS_PER_W, SA), jnp.int32),
            pltpu.VMEM((ROWS_PER_W, SB), jnp.int32),
        ] + [pltpu.VMEM((G * S, D), jnp.float32) for _ in range(NBUF)]
          + [pltpu.SemaphoreType.DMA for _ in range(3 * NBUF)],
        compiler_params=pltpu.CompilerParams(use_tc_tiling_on_sc=False),
    )(embedding_weight, pos2, idxa, idxb)
    return out.reshape(B, S, D)


# speedup vs baseline: 1.7352x; 1.7352x over previous
import jax
import jax.numpy as jnp
from jax import lax
from jax.experimental import pallas as pl
from jax.experimental.pallas import tpu as pltpu
from jax.experimental.pallas import tpu_sc as plsc

VOCAB = 1000000
D = 64
B = 4096
S = 200
N = B * S
C = 128
NC = 2
NS = 16
NW = NC * NS
CHUNKS = N // (NW * C)
PERIOD = 25
NBUF = 4
L = 16


def _body(table_hbm, pos_hbm, idx_hbm, pidx_hbm, out_hbm,
          idx_v, pidx_v, sidx_v, buf0, buf1, buf2, buf3,
          p0, p1, p2, p3, g0, g1, g2, g3, o0, o1, o2, o3):
    bufs = (buf0, buf1, buf2, buf3)
    psems = (p0, p1, p2, p3)
    gsems = (g0, g1, g2, g3)
    osems = (o0, o1, o2, o3)

    cid = lax.axis_index("c")
    sid = lax.axis_index("s")
    wid = sid * NC + cid
    base = wid * CHUNKS

    pltpu.sync_copy(idx_hbm.at[pl.ds(base, CHUNKS)], idx_v)
    pltpu.sync_copy(pidx_hbm, pidx_v)

    def prefill(g, s):
        pat = lax.rem(g, PERIOD)
        return pltpu.make_async_copy(
            pos_hbm.at[pidx_v.at[pat]], bufs[s], psems[s])

    def gather(g, s):
        return pltpu.make_async_copy(
            table_hbm.at[idx_v.at[g]], bufs[s], gsems[s])

    def scatter(g, s):
        return pltpu.make_async_copy(
            bufs[s], out_hbm.at[sidx_v.at[s]], osems[s])

    def build_sidx(g, s):
        start = (base + g) * (2 * C)
        evens = lax.iota(jnp.int32, L) * 2
        for j in range(C // L):
            sidx_v[s, pl.ds(j * L, L)] = evens + (start + 2 * L * j)

    prefill(0, 0).start()

    def step(gg, carry):
        for s in range(NBUF):
            g = gg * NBUF + s
            sn = (s + 1) % NBUF
            sp = (s + NBUF - 1) % NBUF

            prefill(g, s).wait()
            gather(g, s).start(add=True)
            build_sidx(g, s)

            @pl.when(g >= NBUF - 1)
            def _():
                scatter(jnp.maximum(g - (NBUF - 1), 0), sn).wait()

            @pl.when(g + 1 <= CHUNKS - 1)
            def _():
                prefill(g + 1, sn).start()

            @pl.when(g >= 1)
            def _():
                gp = jnp.maximum(g - 1, 0)
                gather(gp, sp).wait()
                scatter(gp, sp).start()
        return carry

    lax.fori_loop(0, CHUNKS // NBUF, step, 0)

    last = CHUNKS - 1
    ls = last % NBUF
    gather(last, ls).wait()
    scatter(last, ls).start()
    for g in range(CHUNKS - NBUF + 1, CHUNKS):
        scatter(g, g % NBUF).wait()


def kernel(x, embedding_weight, positional_weight):
    table3 = embedding_weight[:, None, :]
    pos3 = positional_weight[:S][:, None, :]
    idx2 = (x.astype(jnp.int32) * 2).reshape(N // C, C)
    pidx = ((jnp.arange(PERIOD * C, dtype=jnp.int32) % S) * 2
            ).reshape(PERIOD, C)
    mesh = plsc.VectorSubcoreMesh(core_axis_name="c", subcore_axis_name="s")
    out = pl.kernel(
        _body,
        out_type=jax.ShapeDtypeStruct((N, 1, D), jnp.float32),
        mesh=mesh,
        scratch_types=[
            pltpu.VMEM((CHUNKS, C), jnp.int32),
            pltpu.VMEM((PERIOD, C), jnp.int32),
            pltpu.VMEM((NBUF, C), jnp.int32),
        ] + [pltpu.VMEM((C, 1, D), jnp.float32) for _ in range(NBUF)]
          + [pltpu.SemaphoreType.DMA for _ in range(3 * NBUF)],
    )(table3, pos3, idx2, pidx)
    return out.reshape(B, S, D)


# speedup vs baseline: 2.7645x; 1.5932x over previous
import jax
import jax.numpy as jnp
from jax import lax
from jax.experimental import pallas as pl
from jax.experimental.pallas import tpu as pltpu
from jax.experimental.pallas import tpu_sc as plsc

VOCAB = 1000000
D = 64
B = 4096
S = 200
N = B * S
C = 128
NC = 2
NS = 16
NW = NC * NS
CHUNKS = N // (NW * C)
PERIOD = 25
NBUF = 5
L = 16


def _body(table_hbm, pos_hbm, idx_hbm, pidx_hbm, out_hbm,
          idx_v, pidx_v, sidx_v, buf0, buf1, buf2, buf3, buf4, pos_sh,
          p0, p1, p2, p3, p4, g0, g1, g2, g3, g4, o0, o1, o2, o3, o4):
    bufs = (buf0, buf1, buf2, buf3, buf4)
    psems = (p0, p1, p2, p3, p4)
    gsems = (g0, g1, g2, g3, g4)
    osems = (o0, o1, o2, o3, o4)

    cid = lax.axis_index("c")
    sid = lax.axis_index("s")
    wid = sid * NC + cid
    base = wid * CHUNKS

    @pl.when(sid == 0)
    def _():
        pltpu.sync_copy(pos_hbm, pos_sh)

    pltpu.sync_copy(idx_hbm.at[pl.ds(base, CHUNKS)], idx_v)
    pltpu.sync_copy(pidx_hbm, pidx_v)
    plsc.subcore_barrier()

    def prefill(g, s):
        pat = lax.rem(g, PERIOD)
        return pltpu.make_async_copy(
            pos_sh.at[pidx_v.at[pat]], bufs[s], psems[s])

    def gather(g, s):
        return pltpu.make_async_copy(
            table_hbm.at[idx_v.at[g]], bufs[s], gsems[s])

    def scatter(g, s):
        return pltpu.make_async_copy(
            bufs[s], out_hbm.at[sidx_v.at[s]], osems[s])

    def build_sidx(g, s):
        start = (base + g) * (2 * C)
        evens = lax.iota(jnp.int32, L) * 2
        for j in range(C // L):
            sidx_v[s, pl.ds(j * L, L)] = evens + (start + 2 * L * j)

    prefill(0, 0).start()

    def step(gg, carry):
        for s in range(NBUF):
            g = gg * NBUF + s
            sn = (s + 1) % NBUF
            sp = (s + NBUF - 1) % NBUF

            prefill(g, s).wait()
            gather(g, s).start(add=True)
            build_sidx(g, s)

            @pl.when(g >= NBUF - 1)
            def _():
                scatter(jnp.maximum(g - (NBUF - 1), 0), sn).wait()

            @pl.when(g + 1 <= CHUNKS - 1)
            def _():
                prefill(g + 1, sn).start()

            @pl.when(g >= 1)
            def _():
                gp = jnp.maximum(g - 1, 0)
                gather(gp, sp).wait()
                scatter(gp, sp).start()
        return carry

    lax.fori_loop(0, CHUNKS // NBUF, step, 0)

    last = CHUNKS - 1
    ls = last % NBUF
    gather(last, ls).wait()
    scatter(last, ls).start()
    for g in range(CHUNKS - NBUF + 1, CHUNKS):
        scatter(g, g % NBUF).wait()


def kernel(x, embedding_weight, positional_weight):
    table3 = embedding_weight[:, None, :]
    pos3 = positional_weight[:S][:, None, :]
    idx2 = (x.astype(jnp.int32) * 2).reshape(N // C, C)
    pidx = ((jnp.arange(PERIOD * C, dtype=jnp.int32) % S) * 2
            ).reshape(PERIOD, C)
    mesh = plsc.VectorSubcoreMesh(core_axis_name="c", subcore_axis_name="s")
    out = pl.kernel(
        _body,
        out_type=jax.ShapeDtypeStruct((N, 1, D), jnp.float32),
        mesh=mesh,
        scratch_types=[
            pltpu.VMEM((CHUNKS, C), jnp.int32),
            pltpu.VMEM((PERIOD, C), jnp.int32),
            pltpu.VMEM((NBUF, C), jnp.int32),
        ] + [pltpu.VMEM((C, 1, D), jnp.float32) for _ in range(NBUF)] + [
            pltpu.MemorySpace.VMEM_SHARED((S, 1, D), jnp.float32),
        ] + [pltpu.SemaphoreType.DMA for _ in range(3 * NBUF)],
    )(table3, pos3, idx2, pidx)
    return out.reshape(B, S, D)
